# baseline (speedup 1.0000x reference)
import functools

import jax
import jax.numpy as jnp
from jax import lax
from jax.experimental import pallas as pl
from jax.experimental.pallas import tpu as pltpu
from jax.experimental.pallas import tpu_sc as plsc

B = 16384
D = 64
NF = 5
OUT = 128

_NC = 2
_NS = 16
_NW = _NC * _NS
_BPW = B // _NW
_CHUNK = 128
_NCHUNK = _BPW // _CHUNK
_NIT = _NCHUNK * NF
_NBUF = 5
_L = 16


def _sc_gather_body(samp_hbm, m0, m1, m2, m3, m4, t0, t1, t2, t3, t4,
                    o0, o1, o2, o3, o4, samp_v, idx_v, rows_v, rows64_v,
                    sem_m, sem_g, sem_w):
    wid = lax.axis_index("s") * _NC + lax.axis_index("c")
    base = wid * _BPW
    maps = (m0, m1, m2, m3, m4)
    tabs = (t0, t1, t2, t3, t4)
    outs = (o0, o1, o2, o3, o4)
    pltpu.sync_copy(samp_hbm.at[pl.ds(wid * _NCHUNK, _NCHUNK)], samp_v)
    mdesc = []
    for i in range(_NIT):
        c, f = divmod(i, NF)
        mdesc.append(
            pltpu.async_copy(maps[f].at[samp_v.at[c]], idx_v.at[i], sem_m))

    slot = {}
    k128 = {}
    n128 = 0
    for i in range(_NIT):
        c, f = divmod(i, NF)
        if f == 0:
            slot[i] = rows64_v.at[c]
        else:
            slot[i] = rows_v.at[n128 % _NBUF]
            k128[i] = n128
            n128 += 1
    by_k = {v: k for k, v in k128.items()}

    def _write(j):
        c, f = divmod(j, NF)
        rsl = pl.ds(base + c * _CHUNK, _CHUNK)
        if f == 0:
            return [pltpu.async_copy(slot[j], outs[0].at[rsl, pl.ds(h * D, D)],
                                     sem_w) for h in range(2)]
        return [pltpu.async_copy(slot[j], outs[f].at[rsl], sem_w)]

    gdesc = [None] * _NIT
    wdesc = [None] * _NIT
    waited = [False] * _NIT
    for i in range(_NIT):
        k = k128.get(i)
        if k is not None and k >= _NBUF:
            j = by_k[k - _NBUF]
            for wd in wdesc[j]:
                wd.wait()
            waited[j] = True
        mdesc[i].wait()
        gdesc[i] = pltpu.async_copy(tabs[divmod(i, NF)[1]].at[idx_v.at[i]],
                                    slot[i], sem_g)
        if i >= 1:
            gdesc[i - 1].wait()
            wdesc[i - 1] = _write(i - 1)
    gdesc[_NIT - 1].wait()
    wdesc[_NIT - 1] = _write(_NIT - 1)
    for j in range(_NIT):
        if not waited[j]:
            for wd in wdesc[j]:
                wd.wait()


_SC_MESH = plsc.VectorSubcoreMesh(core_axis_name="c", subcore_axis_name="s")

_sc_gather = functools.partial(
    pl.kernel,
    out_type=[jax.ShapeDtypeStruct((B, 2 * D), jnp.float32)] * NF,
    mesh=_SC_MESH,
    scratch_types=[
        pltpu.VMEM((_NCHUNK, _CHUNK), jnp.int32),
        pltpu.VMEM((_NIT, _CHUNK), jnp.int32),
        pltpu.VMEM((_NBUF, _CHUNK, 2 * D), jnp.float32),
        pltpu.VMEM((_NCHUNK, _CHUNK, D), jnp.float32),
        pltpu.SemaphoreType.DMA,
        pltpu.SemaphoreType.DMA,
        pltpu.SemaphoreType.DMA,
    ],
    compiler_params=pltpu.CompilerParams(use_tc_tiling_on_sc=False,
                                         needs_layout_passes=False),
)(_sc_gather_body)


def _mlp_body(x0, x1, x2, x3, x4, w_ref, b_ref, o_ref):
    x = jnp.concatenate(
        [x0[...], x1[...], x2[...], x3[...], x4[...]], axis=-1)
    h = jnp.dot(x, w_ref[...],
                preferred_element_type=jnp.float32) + b_ref[...]
    o_ref[...] = h * (1.0 / (1.0 + jnp.exp(-h)))


def _mlp(feats, w2, b2d):
    bm = 2048
    in_specs = [pl.BlockSpec((bm, 2 * D), lambda i: (i, 0))
                for _ in range(NF)]
    in_specs += [
        pl.BlockSpec((NF * 2 * D, OUT), lambda i: (0, 0)),
        pl.BlockSpec((1, OUT), lambda i: (0, 0)),
    ]
    return pl.pallas_call(
        _mlp_body,
        grid=(B // bm,),
        in_specs=in_specs,
        out_specs=pl.BlockSpec((bm, OUT), lambda i: (i, 0)),
        out_shape=jax.ShapeDtypeStruct((B, OUT), jnp.float32),
    )(*feats, w2, b2d)


def _padded(table):
    return jnp.pad(table, ((0, 0), (0, D)))


def kernel(sample, map_cat0, map_cat1, map_cat2, map_cat3,
           emb_cat0, emb_cat1, emb_cat2, emb_cat3,
           map_text, text_table, W, b):
    samp2d = sample.astype(jnp.int32).reshape(_NW * _NCHUNK, _CHUNK)
    feats = _sc_gather(
        samp2d,
        map_cat0.astype(jnp.int32), map_cat1.astype(jnp.int32),
        map_cat2.astype(jnp.int32), map_cat3.astype(jnp.int32),
        map_text.astype(jnp.int32),
        emb_cat0, _padded(emb_cat1), _padded(emb_cat2),
        _padded(emb_cat3), _padded(text_table),
    )
    zd = jnp.zeros((D, OUT), W.dtype)
    w2 = jnp.concatenate(
        [m for f in range(NF) for m in (W[f * D:(f + 1) * D], zd)], axis=0)
    return _mlp(feats, w2, b.reshape(1, OUT))


# speedup vs baseline: 1.0935x; 1.0935x over previous
import functools

import jax
import jax.numpy as jnp
from jax import lax
from jax.experimental import pallas as pl
from jax.experimental.pallas import tpu as pltpu
from jax.experimental.pallas import tpu_sc as plsc

B = 16384
D = 64
NF = 5
OUT = 128

_NC = 2
_NS = 16
_NW = _NC * _NS
_BPW = B // _NW
_CHUNK = 128
_NCHUNK = _BPW // _CHUNK
_NIT = _NCHUNK * NF
_NBUF = 6
_L = 16


_SC_MESH = plsc.VectorSubcoreMesh(core_axis_name="c", subcore_axis_name="s")


def _make_sc_gather(nf):
    nit = _NCHUNK * nf
    nbuf = min(_NBUF, nit)

    def body(*refs):
        samp_hbm = refs[0]
        maps = refs[1:1 + nf]
        tabs = refs[1 + nf:1 + 2 * nf]
        outs = refs[1 + 2 * nf:1 + 3 * nf]
        samp_v, idx_v, rows_v, sem_m, sem_g, sem_w = refs[1 + 3 * nf:]
        wid = lax.axis_index("s") * _NC + lax.axis_index("c")
        base = wid * _BPW
        pltpu.sync_copy(samp_hbm.at[pl.ds(wid * _NCHUNK, _NCHUNK)], samp_v)
        mdesc = []
        for i in range(nit):
            c, f = divmod(i, nf)
            mdesc.append(
                pltpu.async_copy(maps[f].at[samp_v.at[c]], idx_v.at[i],
                                 sem_m))

        def _write(j):
            c, f = divmod(j, nf)
            rsl = pl.ds(base + c * _CHUNK, _CHUNK)
            return pltpu.async_copy(rows_v.at[j % nbuf], outs[f].at[rsl],
                                    sem_w)

        gdesc = [None] * nit
        wdesc = [None] * nit
        for i in range(nit):
            if i >= nbuf:
                wdesc[i - nbuf].wait()
            mdesc[i].wait()
            gdesc[i] = pltpu.async_copy(tabs[divmod(i, nf)[1]].at[idx_v.at[i]],
                                        rows_v.at[i % nbuf], sem_g)
            if i >= 1:
                gdesc[i - 1].wait()
                wdesc[i - 1] = _write(i - 1)
        gdesc[nit - 1].wait()
        wdesc[nit - 1] = _write(nit - 1)
        for j in range(nit - nbuf, nit):
            wdesc[j].wait()

    return functools.partial(
        pl.kernel,
        out_type=[jax.ShapeDtypeStruct((B, 2 * D), jnp.float32)] * nf,
        mesh=_SC_MESH,
        scratch_types=[
            pltpu.VMEM((_NCHUNK, _CHUNK), jnp.int32),
            pltpu.VMEM((nit, _CHUNK), jnp.int32),
            pltpu.VMEM((nbuf, _CHUNK, 2 * D), jnp.float32),
            pltpu.SemaphoreType.DMA,
            pltpu.SemaphoreType.DMA,
            pltpu.SemaphoreType.DMA,
        ],
        compiler_params=pltpu.CompilerParams(use_tc_tiling_on_sc=False,
                                             needs_layout_passes=False),
    )(body)


_sc_gather4 = _make_sc_gather(4)
_sc_gather1 = _make_sc_gather(1)


def _mlp_body(x0, x1, x2, x3, x4, w_ref, b_ref, o_ref):
    x = jnp.concatenate(
        [x0[...], x1[...], x2[...], x3[...], x4[...]], axis=-1)
    h = jnp.dot(x, w_ref[...],
                preferred_element_type=jnp.float32) + b_ref[...]
    o_ref[...] = h * (1.0 / (1.0 + jnp.exp(-h)))


def _mlp(feats, w2, b2d):
    bm = 2048
    in_specs = [pl.BlockSpec((bm, 2 * D), lambda i: (i, 0))
                for _ in range(NF)]
    in_specs += [
        pl.BlockSpec((NF * 2 * D, OUT), lambda i: (0, 0)),
        pl.BlockSpec((1, OUT), lambda i: (0, 0)),
    ]
    return pl.pallas_call(
        _mlp_body,
        grid=(B // bm,),
        in_specs=in_specs,
        out_specs=pl.BlockSpec((bm, OUT), lambda i: (i, 0)),
        out_shape=jax.ShapeDtypeStruct((B, OUT), jnp.float32),
    )(*feats, w2, b2d)


def _padded(table):
    return jnp.pad(table, ((0, 0), (0, D)))


def kernel(sample, map_cat0, map_cat1, map_cat2, map_cat3,
           emb_cat0, emb_cat1, emb_cat2, emb_cat3,
           map_text, text_table, W, b):
    samp2d = sample.astype(jnp.int32).reshape(_NW * _NCHUNK, _CHUNK)
    f1, f2, f3, f4 = _sc_gather4(
        samp2d,
        map_cat1.astype(jnp.int32), map_cat2.astype(jnp.int32),
        map_cat3.astype(jnp.int32), map_text.astype(jnp.int32),
        _padded(emb_cat1), _padded(emb_cat2), _padded(emb_cat3),
        _padded(text_table),
    )
    (f0,) = _sc_gather1(samp2d, map_cat0.astype(jnp.int32),
                        _padded(emb_cat0))
    feats = (f0, f1, f2, f3, f4)
    zd = jnp.zeros((D, OUT), W.dtype)
    w2 = jnp.concatenate(
        [m for f in range(NF) for m in (W[f * D:(f + 1) * D], zd)], axis=0)
    return _mlp(feats, w2, b.reshape(1, OUT))


# speedup vs baseline: 1.1531x; 1.0545x over previous
import functools

import jax
import jax.numpy as jnp
from jax import lax
from jax.experimental import pallas as pl
from jax.experimental.pallas import tpu as pltpu
from jax.experimental.pallas import tpu_sc as plsc

B = 16384
D = 64
NF = 5
OUT = 128

_NC = 2
_NS = 16
_NW = _NC * _NS
_BPW = B // _NW
_CHUNK = 128
_NCHUNK = _BPW // _CHUNK
_NIT = _NCHUNK * NF
_NBUF = 6
_L = 16


_SC_MESH = plsc.VectorSubcoreMesh(core_axis_name="c", subcore_axis_name="s")


def _make_sc_gather(nf):
    nit = _NCHUNK * nf
    nbuf = min(_NBUF, nit)

    def body(*refs):
        samp_hbm = refs[0]
        maps = refs[1:1 + nf]
        tabs = refs[1 + nf:1 + 2 * nf]
        outs = refs[1 + 2 * nf:1 + 3 * nf]
        samp_v, idx_v, rows_v, sem_m, sem_g, sem_w = refs[1 + 3 * nf:]
        wid = lax.axis_index("s") * _NC + lax.axis_index("c")
        base = wid * _BPW
        pltpu.sync_copy(samp_hbm.at[pl.ds(wid * _NCHUNK, _NCHUNK)], samp_v)
        mdesc = []
        for i in range(nit):
            c, f = divmod(i, nf)
            mdesc.append(
                pltpu.async_copy(maps[f].at[samp_v.at[c]], idx_v.at[i],
                                 sem_m))

        def _write(j):
            c, f = divmod(j, nf)
            rsl = pl.ds(base + c * _CHUNK, _CHUNK)
            return pltpu.async_copy(rows_v.at[j % nbuf], outs[f].at[rsl],
                                    sem_w)

        gdesc = [None] * nit
        wdesc = [None] * nit
        for i in range(nit):
            if i >= nbuf:
                wdesc[i - nbuf].wait()
            mdesc[i].wait()
            gdesc[i] = pltpu.async_copy(tabs[divmod(i, nf)[1]].at[idx_v.at[i]],
                                        rows_v.at[i % nbuf], sem_g)
            if i >= 1:
                gdesc[i - 1].wait()
                wdesc[i - 1] = _write(i - 1)
        gdesc[nit - 1].wait()
        wdesc[nit - 1] = _write(nit - 1)
        for j in range(nit - nbuf, nit):
            wdesc[j].wait()

    return functools.partial(
        pl.kernel,
        out_type=[jax.ShapeDtypeStruct((B, 2 * D), jnp.float32)] * nf,
        mesh=_SC_MESH,
        scratch_types=[
            pltpu.VMEM((_NCHUNK, _CHUNK), jnp.int32),
            pltpu.VMEM((nit, _CHUNK), jnp.int32),
            pltpu.VMEM((nbuf, _CHUNK, 2 * D), jnp.float32),
            pltpu.SemaphoreType.DMA,
            pltpu.SemaphoreType.DMA,
            pltpu.SemaphoreType.DMA,
        ],
        compiler_params=pltpu.CompilerParams(use_tc_tiling_on_sc=False,
                                             needs_layout_passes=False),
    )(body)


_sc_gather1 = _make_sc_gather(1)


def _sc4_body(samp_hbm, m0, m1, m2, m3, t0, t1, t2, t3,
              oa, ob, samp_v, idx_v, rows_v, sem_m, sem_g, sem_w):
    nf, nit, nbuf = 4, 16, 8
    maps = (m0, m1, m2, m3)
    tabs = (t0, t1, t2, t3)
    wid = lax.axis_index("s") * _NC + lax.axis_index("c")
    base = wid * _BPW
    pltpu.sync_copy(samp_hbm.at[pl.ds(wid * _NCHUNK, _NCHUNK)], samp_v)
    mdesc = []
    for i in range(nit):
        c, f = divmod(i, nf)
        mdesc.append(
            pltpu.async_copy(maps[f].at[samp_v.at[c]], idx_v.at[i], sem_m))

    def _write(j):
        c, f = divmod(j, nf)
        out = (oa, oa, ob, ob)[f]
        rsl = pl.ds(base + c * _CHUNK, _CHUNK)
        return pltpu.async_copy(rows_v.at[j % nbuf],
                                out.at[rsl, pl.ds((f % 2) * D, D)], sem_w)

    gdesc = [None] * nit
    wdesc = [None] * nit
    for i in range(nit):
        if i >= nbuf:
            wdesc[i - nbuf].wait()
        mdesc[i].wait()
        gdesc[i] = pltpu.async_copy(tabs[divmod(i, nf)[1]].at[idx_v.at[i]],
                                    rows_v.at[i % nbuf], sem_g)
        if i >= 1:
            gdesc[i - 1].wait()
            wdesc[i - 1] = _write(i - 1)
    gdesc[nit - 1].wait()
    wdesc[nit - 1] = _write(nit - 1)
    for j in range(nit - nbuf, nit):
        wdesc[j].wait()


_sc_gather4 = functools.partial(
    pl.kernel,
    out_type=[jax.ShapeDtypeStruct((B, 2 * D), jnp.float32)] * 2,
    mesh=_SC_MESH,
    scratch_types=[
        pltpu.VMEM((_NCHUNK, _CHUNK), jnp.int32),
        pltpu.VMEM((16, _CHUNK), jnp.int32),
        pltpu.VMEM((8, _CHUNK, D), jnp.float32),
        pltpu.SemaphoreType.DMA,
        pltpu.SemaphoreType.DMA,
        pltpu.SemaphoreType.DMA,
    ],
    compiler_params=pltpu.CompilerParams(use_tc_tiling_on_sc=False,
                                         needs_layout_passes=False),
)(_sc4_body)


def _mlp_body(x0, x1, x2, w_ref, b_ref, o_ref):
    x = jnp.concatenate([x0[...], x1[...], x2[...]], axis=-1)
    h = jnp.dot(x, w_ref[...],
                preferred_element_type=jnp.float32) + b_ref[...]
    o_ref[...] = h * (1.0 / (1.0 + jnp.exp(-h)))


def _mlp(feats, w2, b2d):
    bm = 2048
    in_specs = [pl.BlockSpec((bm, 2 * D), lambda i: (i, 0))
                for _ in range(3)]
    in_specs += [
        pl.BlockSpec((3 * 2 * D, OUT), lambda i: (0, 0)),
        pl.BlockSpec((1, OUT), lambda i: (0, 0)),
    ]
    return pl.pallas_call(
        _mlp_body,
        grid=(B // bm,),
        in_specs=in_specs,
        out_specs=pl.BlockSpec((bm, OUT), lambda i: (i, 0)),
        out_shape=jax.ShapeDtypeStruct((B, OUT), jnp.float32),
    )(*feats, w2, b2d)


def _padded(table):
    return jnp.pad(table, ((0, 0), (0, D)))


def kernel(sample, map_cat0, map_cat1, map_cat2, map_cat3,
           emb_cat0, emb_cat1, emb_cat2, emb_cat3,
           map_text, text_table, W, b):
    samp2d = sample.astype(jnp.int32).reshape(_NW * _NCHUNK, _CHUNK)
    fa, fb = _sc_gather4(
        samp2d,
        map_cat1.astype(jnp.int32), map_cat2.astype(jnp.int32),
        map_cat3.astype(jnp.int32), map_text.astype(jnp.int32),
        emb_cat1, emb_cat2, emb_cat3, text_table,
    )
    (f0,) = _sc_gather1(samp2d, map_cat0.astype(jnp.int32),
                        _padded(emb_cat0))
    feats = (f0, fa, fb)
    w2 = jnp.concatenate(
        [W[:D], jnp.zeros((D, OUT), W.dtype), W[D:]], axis=0)
    return _mlp(feats, w2, b.reshape(1, OUT))
